# Initial kernel scaffold; baseline (speedup 1.0000x reference)
import jax
import jax.numpy as jnp
from jax.experimental import pallas as pl


def kernel(z, edge_index, feature1, feature2, batch, emb_W, blk_lin_W, blk_lin_b, blk_f1_W1, blk_f1_W2, blk_f2_W1, blk_f2_W2, blk_c1_rel_W, blk_c1_rel_b, blk_c1_root_W, blk_c2_rel_W, blk_c2_rel_b, blk_c2_root_W, blk_lin1_W, blk_lin1_b, blk_lin2_W, blk_lin2_b, blk_lincat_W, blk_lincat_b, blk_norm_w, blk_norm_b, blk_norm_ms, blk_inner_W, blk_inner_b, blk_final_W, blk_final_b, out_lin_W, out_lin_b, enc_W, enc_b, dec_qkv_W, dec_qkv_b, dec_out_W, dec_out_b, dec_ff1_W, dec_ff1_b, dec_ff2_W, dec_ff2_b, dec_ln1_g, dec_ln1_b, dec_ln2_g, dec_ln2_b, head_W, head_b):
    raise NotImplementedError("write your pallas kernel here")


# speedup vs baseline: 1.0861x; 1.0861x over previous
import jax
import jax.numpy as jnp
from jax import lax
from jax.experimental import pallas as pl

N = 10000; E = 160000; G = 50; H = 256; AED = 128
F1D = 12; F2D = 6; NB = 4; NL = 4; OC = 3; FF = 512; NA = 95
GP = 64
NAP = 96
RT = 1000
NT = N // RT
ET = 4000
SEQ = N // G
_F32 = jnp.float32


def _swish(v):
    return v * jax.nn.sigmoid(v)


def _fs(shape):
    return pl.BlockSpec(shape, lambda *_: (0,) * len(shape))


def _rows(block_shape):
    nd = len(block_shape)
    return pl.BlockSpec(block_shape, lambda r: (r,) + (0,) * (nd - 1))


def _dotT(a, b):
    return lax.dot_general(a, b, (((0,), (0,)), ((), ())),
                           preferred_element_type=_F32)


def _embed_body(ohz_ref, emb_ref, o_ref):
    o_ref[...] = _swish(jnp.dot(ohz_ref[...], emb_ref[...],
                                preferred_element_type=_F32))


def _xl_body(x_ref, w_ref, b_ref, o_ref):
    o_ref[...] = _swish(jnp.dot(x_ref[...], w_ref[...],
                                preferred_element_type=_F32) + b_ref[...])


def _feat_body(f1_ref, f2_ref, w11_ref, w12_ref, w21_ref, w22_ref,
               o1_ref, o2_ref):
    m1 = jnp.dot(w11_ref[...], w12_ref[...], preferred_element_type=_F32)
    m2 = jnp.dot(w21_ref[...], w22_ref[...], preferred_element_type=_F32)
    o1_ref[...] = jnp.dot(f1_ref[...], m1, preferred_element_type=_F32)
    o2_ref[...] = jnp.dot(f2_ref[...], m2, preferred_element_type=_F32)


def _dense_body(m1_ref, m2_ref, xl_ref, oh_ref,
                c1w_ref, c1b_ref, c1rw_ref, l1w_ref, l1b_ref,
                c2w_ref, c2b_ref, c2rw_ref, l2w_ref, l2b_ref,
                cw1_ref, cw2_ref, cb_ref, iw_ref, ib_ref,
                h_ref, s1_ref, s2_ref, cnt_ref):
    xl = xl_ref[...]
    h1 = (jnp.dot(m1_ref[...], c1w_ref[...], preferred_element_type=_F32)
          + c1b_ref[...]
          + jnp.dot(xl, c1rw_ref[...], preferred_element_type=_F32))
    h1 = _swish(jnp.dot(h1, l1w_ref[...], preferred_element_type=_F32)
                + l1b_ref[...])
    h2 = (jnp.dot(m2_ref[...], c2w_ref[...], preferred_element_type=_F32)
          + c2b_ref[...]
          + jnp.dot(xl, c2rw_ref[...], preferred_element_type=_F32))
    h2 = _swish(jnp.dot(h2, l2w_ref[...], preferred_element_type=_F32)
                + l2b_ref[...])
    h = (jnp.dot(h1, cw1_ref[...], preferred_element_type=_F32)
         + jnp.dot(h2, cw2_ref[...], preferred_element_type=_F32)
         + cb_ref[...] + xl)
    for j in range(NL):
        h = _swish(jnp.dot(h, iw_ref[j], preferred_element_type=_F32)
                   + ib_ref[j]) + h
    h_ref[...] = h
    oh = oh_ref[...]

    @pl.when(pl.program_id(0) == 0)
    def _init():
        s1_ref[...] = jnp.zeros_like(s1_ref)
        s2_ref[...] = jnp.zeros_like(s2_ref)
        cnt_ref[...] = jnp.zeros_like(cnt_ref)

    s1_ref[...] += _dotT(oh, h)
    s2_ref[...] += _dotT(oh, h * h)
    cnt_ref[...] += _dotT(oh, jnp.ones((oh.shape[0], H), _F32))


def _norm_body(h_ref, oh_ref, s1_ref, s2_ref, cnt_ref,
               nw_ref, nb_ref, ms_ref, fw_ref, fb_ref, o_ref):
    cnt = jnp.maximum(cnt_ref[...], 1.0)
    mean = s1_ref[...] / cnt
    ms = ms_ref[...]
    var = s2_ref[...] / cnt - mean * mean * ms * (2.0 - ms)
    oh = oh_ref[...]
    mean_b = jnp.dot(oh, mean, preferred_element_type=_F32)
    var_b = jnp.dot(oh, var, preferred_element_type=_F32)
    cen = h_ref[...] - mean_b * ms
    hn = nw_ref[...] * cen * lax.rsqrt(var_b + 1e-5) + nb_ref[...]
    o_ref[...] = jnp.dot(hn, fw_ref[...], preferred_element_type=_F32) \
        + fb_ref[...]


def _outenc_body(x_ref, ow_ref, ob_ref, ew_ref, eb_ref, o_ref):
    t = _swish(jnp.dot(x_ref[...], ow_ref[...],
                       preferred_element_type=_F32) + ob_ref[...])
    o_ref[...] = jnp.dot(t, ew_ref[...], preferred_element_type=_F32) \
        + eb_ref[...]


def _ln(x, g, b):
    m = jnp.mean(x, axis=-1, keepdims=True)
    d = x - m
    v = jnp.mean(d * d, axis=-1, keepdims=True)
    return d * lax.rsqrt(v + 1e-5) * g + b


def _dec_body(seq_ref, qw_ref, qb_ref, ow_ref, ob_ref,
              f1w_ref, f1b_ref, f2w_ref, f2b_ref,
              g1_ref, b1_ref, g2_ref, b2_ref, o_ref):
    x0 = seq_ref[0]
    qkv = jnp.dot(x0, qw_ref[0], preferred_element_type=_F32) + qb_ref[0]
    q = qkv[:, :AED]
    k = qkv[:, AED:2 * AED]
    v = qkv[:, 2 * AED:]
    a = lax.dot_general(q, k, (((1,), (1,)), ((), ())),
                        preferred_element_type=_F32) * (1.0 / jnp.sqrt(
                            jnp.float32(AED)))
    a = a - jnp.max(a, axis=-1, keepdims=True)
    a = jnp.exp(a)
    a = a / jnp.sum(a, axis=-1, keepdims=True)
    o = jnp.dot(a, v, preferred_element_type=_F32)
    o = jnp.dot(o, ow_ref[0], preferred_element_type=_F32) + ob_ref[0]
    s = _ln(x0 + o, g1_ref[0], b1_ref[0])
    ffh = jnp.dot(s, f1w_ref[0], preferred_element_type=_F32) + f1b_ref[0]
    ffh = 0.5 * ffh * (1.0 + lax.erf(ffh * (1.0 / jnp.sqrt(jnp.float32(2)))))
    ff = jnp.dot(ffh, f2w_ref[0], preferred_element_type=_F32) + f2b_ref[0]
    o_ref[0, 0] = _ln(s + ff, g2_ref[0], b2_ref[0])


def _final_body(s3_ref, oh_ref, hw_ref, hb_ref, o_ref):
    @pl.when(pl.program_id(0) == 0)
    def _init():
        o_ref[...] = jnp.zeros_like(o_ref)

    oh = oh_ref[...]
    cols = []
    for c in range(OC):
        pn = jnp.dot(s3_ref[c], hw_ref[...],
                     preferred_element_type=_F32) + hb_ref[...]
        cols.append(pn)
    pn3 = jnp.concatenate(cols, axis=1)
    o_ref[...] += _dotT(oh, pn3)


def kernel(z, edge_index, feature1, feature2, batch, emb_W, blk_lin_W,
           blk_lin_b, blk_f1_W1, blk_f1_W2, blk_f2_W1, blk_f2_W2,
           blk_c1_rel_W, blk_c1_rel_b, blk_c1_root_W, blk_c2_rel_W,
           blk_c2_rel_b, blk_c2_root_W, blk_lin1_W, blk_lin1_b,
           blk_lin2_W, blk_lin2_b, blk_lincat_W, blk_lincat_b,
           blk_norm_w, blk_norm_b, blk_norm_ms, blk_inner_W, blk_inner_b,
           blk_final_W, blk_final_b, out_lin_W, out_lin_b, enc_W, enc_b,
           dec_qkv_W, dec_qkv_b, dec_out_W, dec_out_b, dec_ff1_W,
           dec_ff1_b, dec_ff2_W, dec_ff2_b, dec_ln1_g, dec_ln1_b,
           dec_ln2_g, dec_ln2_b, head_W, head_b):
    src = edge_index[0]
    dst = edge_index[1]
    ohz = (z[:, None] == jnp.arange(NAP, dtype=z.dtype)[None, :]
           ).astype(_F32)
    ohb = (batch[:, None] == jnp.arange(GP, dtype=batch.dtype)[None, :]
           ).astype(_F32)
    embp = jnp.pad(emb_W, ((0, NAP - NA), (0, 0)))

    x = pl.pallas_call(
        _embed_body,
        grid=(NT,),
        in_specs=[_rows((RT, NAP)), _fs((NAP, H))],
        out_specs=_rows((RT, H)),
        out_shape=jax.ShapeDtypeStruct((N, H), _F32),
    )(ohz, embp)

    for i in range(NB):
        xl = pl.pallas_call(
            _xl_body,
            grid=(NT,),
            in_specs=[_rows((RT, H)), _fs((H, H)), _fs((1, H))],
            out_specs=_rows((RT, H)),
            out_shape=jax.ShapeDtypeStruct((N, H), _F32),
        )(x, blk_lin_W[i].T, blk_lin_b[i][None, :])

        f1, f2 = pl.pallas_call(
            _feat_body,
            grid=(E // ET,),
            in_specs=[_rows((ET, F1D)), _rows((ET, F2D)),
                      _fs((F1D, H)), _fs((H, H)),
                      _fs((F2D, H)), _fs((H, H))],
            out_specs=[_rows((ET, H)), _rows((ET, H))],
            out_shape=[jax.ShapeDtypeStruct((E, H), _F32),
                       jax.ShapeDtypeStruct((E, H), _F32)],
        )(feature1, feature2, blk_f1_W1[i].T, blk_f1_W2[i].T,
          blk_f2_W1[i].T, blk_f2_W2[i].T)

        xj = jnp.take(xl, src, axis=0)
        m1 = jax.ops.segment_sum(xj * f1, dst, num_segments=N)
        m2 = jax.ops.segment_sum(xj * f2, dst, num_segments=N)

        iwT = jnp.swapaxes(blk_inner_W[i], 1, 2)
        h, s1, s2, cnt = pl.pallas_call(
            _dense_body,
            grid=(NT,),
            in_specs=[_rows((RT, H)), _rows((RT, H)), _rows((RT, H)),
                      _rows((RT, GP)),
                      _fs((H, H)), _fs((1, H)), _fs((H, H)),
                      _fs((H, H)), _fs((1, H)),
                      _fs((H, H)), _fs((1, H)), _fs((H, H)),
                      _fs((H, H)), _fs((1, H)),
                      _fs((H, H)), _fs((H, H)), _fs((1, H)),
                      _fs((NL, H, H)), _fs((NL, 1, H))],
            out_specs=[_rows((RT, H)), _fs((GP, H)), _fs((GP, H)),
                       _fs((GP, H))],
            out_shape=[jax.ShapeDtypeStruct((N, H), _F32),
                       jax.ShapeDtypeStruct((GP, H), _F32),
                       jax.ShapeDtypeStruct((GP, H), _F32),
                       jax.ShapeDtypeStruct((GP, H), _F32)],
        )(m1, m2, xl, ohb,
          blk_c1_rel_W[i].T, blk_c1_rel_b[i][None, :], blk_c1_root_W[i].T,
          blk_lin1_W[i].T, blk_lin1_b[i][None, :],
          blk_c2_rel_W[i].T, blk_c2_rel_b[i][None, :], blk_c2_root_W[i].T,
          blk_lin2_W[i].T, blk_lin2_b[i][None, :],
          blk_lincat_W[i][:, :H].T, blk_lincat_W[i][:, H:].T,
          blk_lincat_b[i][None, :], iwT, blk_inner_b[i][:, None, :])

        x = pl.pallas_call(
            _norm_body,
            grid=(NT,),
            in_specs=[_rows((RT, H)), _rows((RT, GP)),
                      _fs((GP, H)), _fs((GP, H)), _fs((GP, H)),
                      _fs((1, H)), _fs((1, H)), _fs((1, H)),
                      _fs((H, H)), _fs((1, H))],
            out_specs=_rows((RT, H)),
            out_shape=jax.ShapeDtypeStruct((N, H), _F32),
        )(h, ohb, s1, s2, cnt,
          blk_norm_w[i][None, :], blk_norm_b[i][None, :],
          blk_norm_ms[i][None, :],
          blk_final_W[i].T, blk_final_b[i][None, :])

    e = pl.pallas_call(
        _outenc_body,
        grid=(NT,),
        in_specs=[_rows((RT, H)), _fs((H, H)), _fs((1, H)),
                  _fs((H, AED)), _fs((1, AED))],
        out_specs=_rows((RT, AED)),
        out_shape=jax.ShapeDtypeStruct((N, AED), _F32),
    )(x, out_lin_W[0].T, out_lin_b[0][None, :], enc_W.T, enc_b[None, :])

    seq0 = e.reshape(G, SEQ, AED)
    s_all = pl.pallas_call(
        _dec_body,
        grid=(OC, G),
        in_specs=[
            pl.BlockSpec((1, SEQ, AED), lambda c, g: (g, 0, 0)),
            pl.BlockSpec((1, AED, 3 * AED), lambda c, g: (c, 0, 0)),
            pl.BlockSpec((1, 1, 3 * AED), lambda c, g: (c, 0, 0)),
            pl.BlockSpec((1, AED, AED), lambda c, g: (c, 0, 0)),
            pl.BlockSpec((1, 1, AED), lambda c, g: (c, 0, 0)),
            pl.BlockSpec((1, AED, FF), lambda c, g: (c, 0, 0)),
            pl.BlockSpec((1, 1, FF), lambda c, g: (c, 0, 0)),
            pl.BlockSpec((1, FF, AED), lambda c, g: (c, 0, 0)),
            pl.BlockSpec((1, 1, AED), lambda c, g: (c, 0, 0)),
            pl.BlockSpec((1, 1, AED), lambda c, g: (c, 0, 0)),
            pl.BlockSpec((1, 1, AED), lambda c, g: (c, 0, 0)),
            pl.BlockSpec((1, 1, AED), lambda c, g: (c, 0, 0)),
            pl.BlockSpec((1, 1, AED), lambda c, g: (c, 0, 0)),
        ],
        out_specs=pl.BlockSpec((1, 1, SEQ, AED), lambda c, g: (c, g, 0, 0)),
        out_shape=jax.ShapeDtypeStruct((OC, G, SEQ, AED), _F32),
    )(seq0, jnp.swapaxes(dec_qkv_W, 1, 2), dec_qkv_b[:, None, :],
      jnp.swapaxes(dec_out_W, 1, 2), dec_out_b[:, None, :],
      jnp.swapaxes(dec_ff1_W, 1, 2), dec_ff1_b[:, None, :],
      jnp.swapaxes(dec_ff2_W, 1, 2), dec_ff2_b[:, None, :],
      dec_ln1_g[:, None, :], dec_ln1_b[:, None, :],
      dec_ln2_g[:, None, :], dec_ln2_b[:, None, :])

    s3 = s_all.reshape(OC, N, AED)
    out = pl.pallas_call(
        _final_body,
        grid=(NT,),
        in_specs=[pl.BlockSpec((OC, RT, AED), lambda r: (0, r, 0)),
                  _rows((RT, GP)), _fs((AED, 1)), _fs((1, 1))],
        out_specs=_fs((GP, OC)),
        out_shape=jax.ShapeDtypeStruct((GP, OC), _F32),
    )(s3, ohb, head_W.T, head_b[None, :])
    return out[:G]


# speedup vs baseline: 1.1011x; 1.0138x over previous
import jax
import jax.numpy as jnp
from jax import lax
from jax.experimental import pallas as pl

N = 10000; E = 160000; G = 50; H = 256; AED = 128
F1D = 12; F2D = 6; NB = 4; NL = 4; OC = 3; FF = 512; NA = 95
GP = 64
NAP = 96
RT = 1000
NT = N // RT
ET = 4000
SEQ = N // G
_F32 = jnp.float32


def _swish(v):
    return v * jax.nn.sigmoid(v)


def _fs(shape):
    return pl.BlockSpec(shape, lambda *_: (0,) * len(shape))


def _rows(block_shape):
    nd = len(block_shape)
    return pl.BlockSpec(block_shape, lambda r: (r,) + (0,) * (nd - 1))


_HI = lax.Precision.HIGHEST


def _dotT(a, b):
    return lax.dot_general(a, b, (((0,), (0,)), ((), ())),
                           preferred_element_type=_F32, precision=_HI)


def _embed_body(ohz_ref, emb_ref, o_ref):
    o_ref[...] = _swish(jnp.dot(ohz_ref[...], emb_ref[...],
                                preferred_element_type=_F32,
                                precision=_HI))


def _xl_body(x_ref, w_ref, b_ref, o_ref):
    o_ref[...] = _swish(jnp.dot(x_ref[...], w_ref[...],
                                preferred_element_type=_F32) + b_ref[...])


def _featprod_body(f1_ref, f2_ref, xj_ref, w11_ref, w12_ref, w21_ref,
                   w22_ref, p_ref):
    m1 = jnp.dot(w11_ref[...], w12_ref[...], preferred_element_type=_F32,
                 precision=_HI)
    m2 = jnp.dot(w21_ref[...], w22_ref[...], preferred_element_type=_F32,
                 precision=_HI)
    xj = xj_ref[...]
    p_ref[:, :H] = xj * jnp.dot(f1_ref[...], m1,
                                preferred_element_type=_F32)
    p_ref[:, H:] = xj * jnp.dot(f2_ref[...], m2,
                                preferred_element_type=_F32)


def _dense_body(m_ref, xl_ref, oh_ref,
                c1w_ref, c1b_ref, c1rw_ref, l1w_ref, l1b_ref,
                c2w_ref, c2b_ref, c2rw_ref, l2w_ref, l2b_ref,
                cw1_ref, cw2_ref, cb_ref, iw_ref, ib_ref,
                h_ref, s1_ref, cnt_ref):
    xl = xl_ref[...]
    h1 = (jnp.dot(m_ref[:, :H], c1w_ref[...], preferred_element_type=_F32)
          + c1b_ref[...]
          + jnp.dot(xl, c1rw_ref[...], preferred_element_type=_F32))
    h1 = _swish(jnp.dot(h1, l1w_ref[...], preferred_element_type=_F32)
                + l1b_ref[...])
    h2 = (jnp.dot(m_ref[:, H:], c2w_ref[...], preferred_element_type=_F32)
          + c2b_ref[...]
          + jnp.dot(xl, c2rw_ref[...], preferred_element_type=_F32))
    h2 = _swish(jnp.dot(h2, l2w_ref[...], preferred_element_type=_F32)
                + l2b_ref[...])
    h = (jnp.dot(h1, cw1_ref[...], preferred_element_type=_F32)
         + jnp.dot(h2, cw2_ref[...], preferred_element_type=_F32)
         + cb_ref[...] + xl)
    for j in range(NL):
        h = _swish(jnp.dot(h, iw_ref[j], preferred_element_type=_F32)
                   + ib_ref[j]) + h
    h_ref[...] = h
    oh = oh_ref[...]

    @pl.when(pl.program_id(0) == 0)
    def _init():
        s1_ref[...] = jnp.zeros_like(s1_ref)
        cnt_ref[...] = jnp.zeros_like(cnt_ref)

    s1_ref[...] += _dotT(oh, h)
    cnt_ref[...] += _dotT(oh, jnp.ones((oh.shape[0], H), _F32))


def _normA_body(h_ref, oh_ref, s1_ref, cnt_ref, ms_ref,
                cen_ref, sc2_ref):
    cnt = jnp.maximum(cnt_ref[...], 1.0)
    mean = s1_ref[...] / cnt
    oh = oh_ref[...]
    mean_b = jnp.dot(oh, mean, preferred_element_type=_F32, precision=_HI)
    cen = h_ref[...] - mean_b * ms_ref[...]
    cen_ref[...] = cen

    @pl.when(pl.program_id(0) == 0)
    def _init():
        sc2_ref[...] = jnp.zeros_like(sc2_ref)

    sc2_ref[...] += _dotT(oh, cen * cen)


def _normB_body(cen_ref, oh_ref, sc2_ref, cnt_ref,
                nw_ref, nb_ref, fw_ref, fb_ref, o_ref):
    var = sc2_ref[...] / jnp.maximum(cnt_ref[...], 1.0)
    oh = oh_ref[...]
    var_b = jnp.dot(oh, var, preferred_element_type=_F32, precision=_HI)
    cen = cen_ref[...]
    hn = nw_ref[...] * cen * lax.rsqrt(var_b + 1e-5) + nb_ref[...]
    o_ref[...] = jnp.dot(hn, fw_ref[...], preferred_element_type=_F32) \
        + fb_ref[...]


def _outenc_body(x_ref, ow_ref, ob_ref, ew_ref, eb_ref, o_ref):
    t = _swish(jnp.dot(x_ref[...], ow_ref[...],
                       preferred_element_type=_F32) + ob_ref[...])
    o_ref[...] = jnp.dot(t, ew_ref[...], preferred_element_type=_F32) \
        + eb_ref[...]


def _ln(x, g, b):
    m = jnp.mean(x, axis=-1, keepdims=True)
    d = x - m
    v = jnp.mean(d * d, axis=-1, keepdims=True)
    return d * lax.rsqrt(v + 1e-5) * g + b


def _dec_body(seq_ref, qw_ref, qb_ref, ow_ref, ob_ref,
              f1w_ref, f1b_ref, f2w_ref, f2b_ref,
              g1_ref, b1_ref, g2_ref, b2_ref, o_ref):
    x0 = seq_ref[0]
    qkv = jnp.dot(x0, qw_ref[0], preferred_element_type=_F32) + qb_ref[0]
    q = qkv[:, :AED]
    k = qkv[:, AED:2 * AED]
    v = qkv[:, 2 * AED:]
    a = lax.dot_general(q, k, (((1,), (1,)), ((), ())),
                        preferred_element_type=_F32) * (1.0 / jnp.sqrt(
                            jnp.float32(AED)))
    a = a - jnp.max(a, axis=-1, keepdims=True)
    a = jnp.exp(a)
    a = a / jnp.sum(a, axis=-1, keepdims=True)
    o = jnp.dot(a, v, preferred_element_type=_F32)
    o = jnp.dot(o, ow_ref[0], preferred_element_type=_F32) + ob_ref[0]
    s = _ln(x0 + o, g1_ref[0], b1_ref[0])
    ffh = jnp.dot(s, f1w_ref[0], preferred_element_type=_F32) + f1b_ref[0]
    ffh = 0.5 * ffh * (1.0 + lax.erf(ffh * (1.0 / jnp.sqrt(jnp.float32(2)))))
    ff = jnp.dot(ffh, f2w_ref[0], preferred_element_type=_F32) + f2b_ref[0]
    o_ref[0, 0] = _ln(s + ff, g2_ref[0], b2_ref[0])


def _final_body(s3_ref, oh_ref, hw_ref, hb_ref, o_ref):
    @pl.when(pl.program_id(0) == 0)
    def _init():
        o_ref[...] = jnp.zeros_like(o_ref)

    oh = oh_ref[...]
    cols = []
    for c in range(OC):
        pn = jnp.dot(s3_ref[c], hw_ref[...],
                     preferred_element_type=_F32) + hb_ref[...]
        cols.append(pn)
    pn3 = jnp.concatenate(cols, axis=1)
    o_ref[...] += _dotT(oh, pn3)


def kernel(z, edge_index, feature1, feature2, batch, emb_W, blk_lin_W,
           blk_lin_b, blk_f1_W1, blk_f1_W2, blk_f2_W1, blk_f2_W2,
           blk_c1_rel_W, blk_c1_rel_b, blk_c1_root_W, blk_c2_rel_W,
           blk_c2_rel_b, blk_c2_root_W, blk_lin1_W, blk_lin1_b,
           blk_lin2_W, blk_lin2_b, blk_lincat_W, blk_lincat_b,
           blk_norm_w, blk_norm_b, blk_norm_ms, blk_inner_W, blk_inner_b,
           blk_final_W, blk_final_b, out_lin_W, out_lin_b, enc_W, enc_b,
           dec_qkv_W, dec_qkv_b, dec_out_W, dec_out_b, dec_ff1_W,
           dec_ff1_b, dec_ff2_W, dec_ff2_b, dec_ln1_g, dec_ln1_b,
           dec_ln2_g, dec_ln2_b, head_W, head_b):
    src = edge_index[0]
    dst = edge_index[1]
    ohz = (z[:, None] == jnp.arange(NAP, dtype=z.dtype)[None, :]
           ).astype(_F32)
    ohb = (batch[:, None] == jnp.arange(GP, dtype=batch.dtype)[None, :]
           ).astype(_F32)
    embp = jnp.pad(emb_W, ((0, NAP - NA), (0, 0)))

    x = pl.pallas_call(
        _embed_body,
        grid=(NT,),
        in_specs=[_rows((RT, NAP)), _fs((NAP, H))],
        out_specs=_rows((RT, H)),
        out_shape=jax.ShapeDtypeStruct((N, H), _F32),
    )(ohz, embp)

    for i in range(NB):
        xl = pl.pallas_call(
            _xl_body,
            grid=(NT,),
            in_specs=[_rows((RT, H)), _fs((H, H)), _fs((1, H))],
            out_specs=_rows((RT, H)),
            out_shape=jax.ShapeDtypeStruct((N, H), _F32),
        )(x, blk_lin_W[i].T, blk_lin_b[i][None, :])

        xj = jnp.take(xl, src, axis=0)
        p = pl.pallas_call(
            _featprod_body,
            grid=(E // ET,),
            in_specs=[_rows((ET, F1D)), _rows((ET, F2D)), _rows((ET, H)),
                      _fs((F1D, H)), _fs((H, H)),
                      _fs((F2D, H)), _fs((H, H))],
            out_specs=_rows((ET, 2 * H)),
            out_shape=jax.ShapeDtypeStruct((E, 2 * H), _F32),
        )(feature1, feature2, xj, blk_f1_W1[i].T, blk_f1_W2[i].T,
          blk_f2_W1[i].T, blk_f2_W2[i].T)

        m = jax.ops.segment_sum(p, dst, num_segments=N)

        iwT = jnp.swapaxes(blk_inner_W[i], 1, 2)
        h, s1, cnt = pl.pallas_call(
            _dense_body,
            grid=(NT,),
            in_specs=[_rows((RT, 2 * H)), _rows((RT, H)),
                      _rows((RT, GP)),
                      _fs((H, H)), _fs((1, H)), _fs((H, H)),
                      _fs((H, H)), _fs((1, H)),
                      _fs((H, H)), _fs((1, H)), _fs((H, H)),
                      _fs((H, H)), _fs((1, H)),
                      _fs((H, H)), _fs((H, H)), _fs((1, H)),
                      _fs((NL, H, H)), _fs((NL, 1, H))],
            out_specs=[_rows((RT, H)), _fs((GP, H)), _fs((GP, H))],
            out_shape=[jax.ShapeDtypeStruct((N, H), _F32),
                       jax.ShapeDtypeStruct((GP, H), _F32),
                       jax.ShapeDtypeStruct((GP, H), _F32)],
        )(m, xl, ohb,
          blk_c1_rel_W[i].T, blk_c1_rel_b[i][None, :], blk_c1_root_W[i].T,
          blk_lin1_W[i].T, blk_lin1_b[i][None, :],
          blk_c2_rel_W[i].T, blk_c2_rel_b[i][None, :], blk_c2_root_W[i].T,
          blk_lin2_W[i].T, blk_lin2_b[i][None, :],
          blk_lincat_W[i][:, :H].T, blk_lincat_W[i][:, H:].T,
          blk_lincat_b[i][None, :], iwT, blk_inner_b[i][:, None, :])

        cen, sc2 = pl.pallas_call(
            _normA_body,
            grid=(NT,),
            in_specs=[_rows((RT, H)), _rows((RT, GP)),
                      _fs((GP, H)), _fs((GP, H)), _fs((1, H))],
            out_specs=[_rows((RT, H)), _fs((GP, H))],
            out_shape=[jax.ShapeDtypeStruct((N, H), _F32),
                       jax.ShapeDtypeStruct((GP, H), _F32)],
        )(h, ohb, s1, cnt, blk_norm_ms[i][None, :])

        x = pl.pallas_call(
            _normB_body,
            grid=(NT,),
            in_specs=[_rows((RT, H)), _rows((RT, GP)),
                      _fs((GP, H)), _fs((GP, H)),
                      _fs((1, H)), _fs((1, H)),
                      _fs((H, H)), _fs((1, H))],
            out_specs=_rows((RT, H)),
            out_shape=jax.ShapeDtypeStruct((N, H), _F32),
        )(cen, ohb, sc2, cnt,
          blk_norm_w[i][None, :], blk_norm_b[i][None, :],
          blk_final_W[i].T, blk_final_b[i][None, :])

    e = pl.pallas_call(
        _outenc_body,
        grid=(NT,),
        in_specs=[_rows((RT, H)), _fs((H, H)), _fs((1, H)),
                  _fs((H, AED)), _fs((1, AED))],
        out_specs=_rows((RT, AED)),
        out_shape=jax.ShapeDtypeStruct((N, AED), _F32),
    )(x, out_lin_W[0].T, out_lin_b[0][None, :], enc_W.T, enc_b[None, :])

    seq0 = e.reshape(G, SEQ, AED)
    s_all = pl.pallas_call(
        _dec_body,
        grid=(OC, G),
        in_specs=[
            pl.BlockSpec((1, SEQ, AED), lambda c, g: (g, 0, 0)),
            pl.BlockSpec((1, AED, 3 * AED), lambda c, g: (c, 0, 0)),
            pl.BlockSpec((1, 1, 3 * AED), lambda c, g: (c, 0, 0)),
            pl.BlockSpec((1, AED, AED), lambda c, g: (c, 0, 0)),
            pl.BlockSpec((1, 1, AED), lambda c, g: (c, 0, 0)),
            pl.BlockSpec((1, AED, FF), lambda c, g: (c, 0, 0)),
            pl.BlockSpec((1, 1, FF), lambda c, g: (c, 0, 0)),
            pl.BlockSpec((1, FF, AED), lambda c, g: (c, 0, 0)),
            pl.BlockSpec((1, 1, AED), lambda c, g: (c, 0, 0)),
            pl.BlockSpec((1, 1, AED), lambda c, g: (c, 0, 0)),
            pl.BlockSpec((1, 1, AED), lambda c, g: (c, 0, 0)),
            pl.BlockSpec((1, 1, AED), lambda c, g: (c, 0, 0)),
            pl.BlockSpec((1, 1, AED), lambda c, g: (c, 0, 0)),
        ],
        out_specs=pl.BlockSpec((1, 1, SEQ, AED), lambda c, g: (c, g, 0, 0)),
        out_shape=jax.ShapeDtypeStruct((OC, G, SEQ, AED), _F32),
    )(seq0, jnp.swapaxes(dec_qkv_W, 1, 2), dec_qkv_b[:, None, :],
      jnp.swapaxes(dec_out_W, 1, 2), dec_out_b[:, None, :],
      jnp.swapaxes(dec_ff1_W, 1, 2), dec_ff1_b[:, None, :],
      jnp.swapaxes(dec_ff2_W, 1, 2), dec_ff2_b[:, None, :],
      dec_ln1_g[:, None, :], dec_ln1_b[:, None, :],
      dec_ln2_g[:, None, :], dec_ln2_b[:, None, :])

    s3 = s_all.reshape(OC, N, AED)
    out = pl.pallas_call(
        _final_body,
        grid=(NT,),
        in_specs=[pl.BlockSpec((OC, RT, AED), lambda r: (0, r, 0)),
                  _rows((RT, GP)), _fs((AED, 1)), _fs((1, 1))],
        out_specs=_fs((GP, OC)),
        out_shape=jax.ShapeDtypeStruct((GP, OC), _F32),
    )(s3, ohb, head_W.T, head_b[None, :])
    return out[:G]
